# Initial kernel scaffold; baseline (speedup 1.0000x reference)
import jax
import jax.numpy as jnp
from jax.experimental import pallas as pl


def kernel(q_seq, l_seq, concepts, operate, btype, text, text1, text2, params, graph):
    raise NotImplementedError("write your pallas kernel here")


# speedup vs baseline: 1.0412x; 1.0412x over previous
import functools

import jax
import jax.numpy as jnp
from jax.experimental import pallas as pl
from jax.experimental.pallas import tpu as pltpu

QN, CN, BERT, D, BS, L, KC, NL = 20000, 2000, 768, 128, 64, 200, 4, 2
H = 2 * D


def _mm_kernel(x_ref, w_ref, b_ref, o_ref):
    o_ref[...] = (
        jnp.dot(x_ref[...], w_ref[...], preferred_element_type=jnp.float32)
        + b_ref[...]
    )


def _matmul_bias(x, w, b, bm):
    m, k = x.shape
    n = w.shape[1]
    assert m % bm == 0, (m, bm)
    return pl.pallas_call(
        _mm_kernel,
        grid=(m // bm,),
        in_specs=[
            pl.BlockSpec((bm, k), lambda i: (i, 0)),
            pl.BlockSpec((k, n), lambda i: (0, 0)),
            pl.BlockSpec((1, n), lambda i: (0, 0)),
        ],
        out_specs=pl.BlockSpec((bm, n), lambda i: (i, 0)),
        out_shape=jax.ShapeDtypeStruct((m, n), jnp.float32),
    )(x, w, b.reshape(1, n))


def _gru_pred_kernel(gi_ref, eq_ref, ec_ref, whh_ref, bhh_ref,
                     w1h_ref, w1q_ref, w1c_ref, b1_ref, w2_ref, b2_ref,
                     o_ref, h_ref):
    t = pl.program_id(0)

    @pl.when(t == 0)
    def _():
        h_ref[...] = jnp.zeros_like(h_ref)

    h = h_ref[...]
    eq = eq_ref[0]
    ec = ec_ref[0]

    px = jnp.tanh(
        jnp.dot(h, w1h_ref[...], preferred_element_type=jnp.float32)
        + jnp.dot(eq, w1q_ref[...], preferred_element_type=jnp.float32)
        + jnp.dot(ec, w1c_ref[...], preferred_element_type=jnp.float32)
        + b1_ref[...]
    )
    logit = jnp.sum(px * w2_ref[...], axis=-1) + b2_ref[0, 0]
    o_ref[0, 0] = jax.nn.sigmoid(logit)

    gi = gi_ref[0]
    gh = jnp.dot(h, whh_ref[...], preferred_element_type=jnp.float32) + bhh_ref[...]
    r = jax.nn.sigmoid(gi[:, :H] + gh[:, :H])
    z = jax.nn.sigmoid(gi[:, H:2 * H] + gh[:, H:2 * H])
    nn = jnp.tanh(gi[:, 2 * H:] + r * gh[:, 2 * H:])
    h_ref[...] = (1.0 - z) * nn + z * h


def _gru_predict(gi, eq, ec, whh_t, bhh, w1h, w1q, w1c, b1, w2row, b2):
    out = pl.pallas_call(
        _gru_pred_kernel,
        grid=(L,),
        in_specs=[
            pl.BlockSpec((1, BS, 3 * H), lambda t: (t, 0, 0)),
            pl.BlockSpec((1, BS, D), lambda t: (t, 0, 0)),
            pl.BlockSpec((1, BS, D), lambda t: (t, 0, 0)),
            pl.BlockSpec((H, 3 * H), lambda t: (0, 0)),
            pl.BlockSpec((1, 3 * H), lambda t: (0, 0)),
            pl.BlockSpec((H, D), lambda t: (0, 0)),
            pl.BlockSpec((D, D), lambda t: (0, 0)),
            pl.BlockSpec((D, D), lambda t: (0, 0)),
            pl.BlockSpec((1, D), lambda t: (0, 0)),
            pl.BlockSpec((1, D), lambda t: (0, 0)),
            pl.BlockSpec((1, 1), lambda t: (0, 0)),
        ],
        out_specs=pl.BlockSpec((1, 1, BS), lambda t: (t, 0, 0)),
        out_shape=jax.ShapeDtypeStruct((L, 1, BS), jnp.float32),
        scratch_shapes=[pltpu.VMEM((BS, H), jnp.float32)],
    )(gi, eq, ec, whh_t, bhh.reshape(1, -1), w1h, w1q, w1c,
      b1.reshape(1, -1), w2row, b2.reshape(1, 1))
    return out.reshape(L, BS).T


def _gat(x_src, x_dst, src, dst, p, pre, n_dst):
    xs = x_src @ p[pre + '_Wsrc']
    xd = x_dst @ p[pre + '_Wdst']
    e = jax.nn.leaky_relu((xs @ p[pre + '_asrc'])[src] + (xd @ p[pre + '_adst'])[dst], 0.2)
    m = jax.ops.segment_max(e, dst, num_segments=n_dst)
    m = jnp.where(jnp.isfinite(m), m, 0.0)
    ex = jnp.exp(e - m[dst])
    s = jax.ops.segment_sum(ex, dst, num_segments=n_dst)
    alpha = ex / (s[dst] + 1e-16)
    return jax.ops.segment_sum(alpha[:, None] * xs[src], dst, num_segments=n_dst) + p[pre + '_b']


def _gnn(p, g):
    qx, cx = p['q_emb'], p['c_emb']
    for i in range(NL):
        c_new = (_gat(cx, cx, g['cc_src'], g['cc_dst'], p, 'cc%d' % i, CN)
                 + _gat(qx, cx, g['qc_src'], g['qc_dst'], p, 'qc%d' % i, CN)
                 + cx @ p['linWc%d' % i] + p['linbc%d' % i])
        q_new = (_gat(cx, qx, g['cq_src'], g['cq_dst'], p, 'cq%d' % i, QN)
                 + qx @ p['linWq%d' % i] + p['linbq%d' % i])
        cx = jax.nn.relu(c_new)
        qx = jax.nn.relu(q_new)
    return qx, cx


def kernel(q_seq, l_seq, concepts, operate, btype, text, text1, text2, params, graph):
    p = params
    qx, cx = _gnn(p, graph)

    q_tab = jnp.concatenate([jnp.zeros((1, D), qx.dtype), qx], 0)
    c_tab = jnp.concatenate([jnp.zeros((1, D), cx.dtype), cx], 0)
    e_q = q_tab[q_seq]
    e_c_m = c_tab[concepts]
    filt = (concepts != 0).astype(jnp.float32)
    fs = filt.sum(-1)
    div = jnp.where(fs == 0, 1.0, fs)[..., None]
    e_c = e_c_m.sum(-2) / div

    op = operate.astype(jnp.float32)[..., None]
    v = jnp.concatenate([e_c * op, e_c * (1.0 - op)], -1)

    gi = _matmul_bias(v.reshape(BS * L, H), p['gru_Wih'].T, p['gru_bih'], 512)
    gi = gi.reshape(BS, L, 3 * H)

    gi_t = jnp.transpose(gi, (1, 0, 2))
    eq_t = jnp.transpose(e_q, (1, 0, 2))
    ec_t = jnp.transpose(e_c, (1, 0, 2))

    w1 = p['predW1']
    prob = _gru_predict(
        gi_t, eq_t, ec_t,
        p['gru_Whh'].T, p['gru_bhh'],
        w1[:H], w1[H:H + D], w1[H + D:], p['predb1'],
        p['predW2'].reshape(1, D), p['predb2'],
    )
    return prob


# speedup vs baseline: 1.3148x; 1.2627x over previous
import functools

import jax
import jax.numpy as jnp
from jax.experimental import pallas as pl
from jax.experimental.pallas import tpu as pltpu

QN, CN, BERT, D, BS, L, KC, NL = 20000, 2000, 768, 128, 64, 200, 4, 2
H = 2 * D


def _mm_kernel(x_ref, w_ref, b_ref, o_ref):
    o_ref[...] = (
        jnp.dot(x_ref[...], w_ref[...], preferred_element_type=jnp.float32)
        + b_ref[...]
    )


def _matmul_bias(x, w, b, bm):
    m, k = x.shape
    n = w.shape[1]
    assert m % bm == 0, (m, bm)
    return pl.pallas_call(
        _mm_kernel,
        grid=(m // bm,),
        in_specs=[
            pl.BlockSpec((bm, k), lambda i: (i, 0)),
            pl.BlockSpec((k, n), lambda i: (0, 0)),
            pl.BlockSpec((1, n), lambda i: (0, 0)),
        ],
        out_specs=pl.BlockSpec((bm, n), lambda i: (i, 0)),
        out_shape=jax.ShapeDtypeStruct((m, n), jnp.float32),
    )(x, w, b.reshape(1, n))


def _gru_pred_kernel(gi_ref, eq_ref, ec_ref, whh_ref, bhh_ref,
                     w1h_ref, w1q_ref, w1c_ref, b1_ref, w2_ref, b2_ref,
                     o_ref, h_ref):
    t = pl.program_id(0)

    @pl.when(t == 0)
    def _():
        h_ref[...] = jnp.zeros_like(h_ref)

    h = h_ref[...]
    eq = eq_ref[0]
    ec = ec_ref[0]

    px = jnp.tanh(
        jnp.dot(h, w1h_ref[...], preferred_element_type=jnp.float32)
        + jnp.dot(eq, w1q_ref[...], preferred_element_type=jnp.float32)
        + jnp.dot(ec, w1c_ref[...], preferred_element_type=jnp.float32)
        + b1_ref[...]
    )
    logit = jnp.sum(px * w2_ref[...], axis=-1) + b2_ref[0, 0]
    o_ref[0, 0] = jax.nn.sigmoid(logit)

    gi = gi_ref[0]
    gh = jnp.dot(h, whh_ref[...], preferred_element_type=jnp.float32) + bhh_ref[...]
    r = jax.nn.sigmoid(gi[:, :H] + gh[:, :H])
    z = jax.nn.sigmoid(gi[:, H:2 * H] + gh[:, H:2 * H])
    nn = jnp.tanh(gi[:, 2 * H:] + r * gh[:, 2 * H:])
    h_ref[...] = (1.0 - z) * nn + z * h


def _gru_predict(gi, eq, ec, whh_t, bhh, w1h, w1q, w1c, b1, w2row, b2):
    out = pl.pallas_call(
        _gru_pred_kernel,
        grid=(L,),
        in_specs=[
            pl.BlockSpec((1, BS, 3 * H), lambda t: (t, 0, 0)),
            pl.BlockSpec((1, BS, D), lambda t: (t, 0, 0)),
            pl.BlockSpec((1, BS, D), lambda t: (t, 0, 0)),
            pl.BlockSpec((H, 3 * H), lambda t: (0, 0)),
            pl.BlockSpec((1, 3 * H), lambda t: (0, 0)),
            pl.BlockSpec((H, D), lambda t: (0, 0)),
            pl.BlockSpec((D, D), lambda t: (0, 0)),
            pl.BlockSpec((D, D), lambda t: (0, 0)),
            pl.BlockSpec((1, D), lambda t: (0, 0)),
            pl.BlockSpec((1, D), lambda t: (0, 0)),
            pl.BlockSpec((1, 1), lambda t: (0, 0)),
        ],
        out_specs=pl.BlockSpec((1, 1, BS), lambda t: (t, 0, 0)),
        out_shape=jax.ShapeDtypeStruct((L, 1, BS), jnp.float32),
        scratch_shapes=[pltpu.VMEM((BS, H), jnp.float32)],
    )(gi, eq, ec, whh_t, bhh.reshape(1, -1), w1h, w1q, w1c,
      b1.reshape(1, -1), w2row, b2.reshape(1, 1))
    return out.reshape(L, BS).T


def _proj_kernel(x_ref, wm_ref, ws_ref, om_ref, os_ref):
    x = x_ref[...]
    om_ref[...] = jnp.dot(x, wm_ref[...], preferred_element_type=jnp.float32)
    os_ref[...] = jnp.dot(x, ws_ref[...], preferred_element_type=jnp.float32)


def _project(x, wm, ws, bm):
    m, k = x.shape
    nm, ns = wm.shape[1], ws.shape[1]
    assert m % bm == 0
    return pl.pallas_call(
        _proj_kernel,
        grid=(m // bm,),
        in_specs=[
            pl.BlockSpec((bm, k), lambda i: (i, 0)),
            pl.BlockSpec((k, nm), lambda i: (0, 0)),
            pl.BlockSpec((k, ns), lambda i: (0, 0)),
        ],
        out_specs=[
            pl.BlockSpec((bm, nm), lambda i: (i, 0)),
            pl.BlockSpec((bm, ns), lambda i: (i, 0)),
        ],
        out_shape=[
            jax.ShapeDtypeStruct((m, nm), jnp.float32),
            jax.ShapeDtypeStruct((m, ns), jnp.float32),
        ],
    )(x, wm, ws)


def _gat_agg(xs, es, ed, src, dst, n_dst):
    e = jax.nn.leaky_relu(es[src] + ed[dst], 0.2)
    m = jax.ops.segment_max(e, dst, num_segments=n_dst)
    m = jnp.where(jnp.isfinite(m), m, 0.0)
    ex = jnp.exp(e - m[dst])
    s = jax.ops.segment_sum(ex, dst, num_segments=n_dst)
    num = jax.ops.segment_sum(ex[:, None] * xs[src], dst, num_segments=n_dst)
    return num, s


def _gnn(p, g):
    qx, cx = p['q_emb'], p['c_emb']
    eps = 1e-16
    for i in range(NL):
        wm_q = jnp.concatenate([p['qc%d_Wsrc' % i], p['linWq%d' % i]], 1)
        ws_q = jnp.stack([
            p['qc%d_Wsrc' % i] @ p['qc%d_asrc' % i],
            p['cq%d_Wdst' % i] @ p['cq%d_adst' % i],
        ], 1)
        ws_q = jnp.pad(ws_q, ((0, 0), (0, 6)))
        wm_c = jnp.concatenate(
            [p['cc%d_Wsrc' % i], p['cq%d_Wsrc' % i], p['linWc%d' % i]], 1)
        ws_c = jnp.stack([
            p['cc%d_Wsrc' % i] @ p['cc%d_asrc' % i],
            p['cc%d_Wdst' % i] @ p['cc%d_adst' % i],
            p['qc%d_Wdst' % i] @ p['qc%d_adst' % i],
            p['cq%d_Wsrc' % i] @ p['cq%d_asrc' % i],
        ], 1)
        ws_c = jnp.pad(ws_c, ((0, 0), (0, 4)))

        pq_m, pq_s = _project(qx, wm_q, ws_q, 400)
        pc_m, pc_s = _project(cx, wm_c, ws_c, 400)
        qc_xs, lin_q = pq_m[:, :D], pq_m[:, D:]
        es_qc, ed_cq = pq_s[:, 0], pq_s[:, 1]
        cc_xs, cq_xs, lin_c = pc_m[:, :D], pc_m[:, D:2 * D], pc_m[:, 2 * D:]
        es_cc, ed_cc = pc_s[:, 0], pc_s[:, 1]
        ed_qc, es_cq = pc_s[:, 2], pc_s[:, 3]

        n_cc, s_cc = _gat_agg(cc_xs, es_cc, ed_cc, g['cc_src'], g['cc_dst'], CN)
        n_qc, s_qc = _gat_agg(qc_xs, es_qc, ed_qc, g['qc_src'], g['qc_dst'], CN)
        n_cq, s_cq = _gat_agg(cq_xs, es_cq, ed_cq, g['cq_src'], g['cq_dst'], QN)

        cx = jax.nn.relu(n_cc / (s_cc[:, None] + eps) + p['cc%d_b' % i]
                         + n_qc / (s_qc[:, None] + eps) + p['qc%d_b' % i]
                         + lin_c + p['linbc%d' % i])
        qx = jax.nn.relu(n_cq / (s_cq[:, None] + eps) + p['cq%d_b' % i]
                         + lin_q + p['linbq%d' % i])
    return qx, cx


def kernel(q_seq, l_seq, concepts, operate, btype, text, text1, text2, params, graph):
    p = params
    qx, cx = _gnn(p, graph)

    q_tab = jnp.concatenate([jnp.zeros((1, D), qx.dtype), qx], 0)
    c_tab = jnp.concatenate([jnp.zeros((1, D), cx.dtype), cx], 0)
    e_q = q_tab[q_seq]
    e_c_m = c_tab[concepts]
    filt = (concepts != 0).astype(jnp.float32)
    fs = filt.sum(-1)
    div = jnp.where(fs == 0, 1.0, fs)[..., None]
    e_c = e_c_m.sum(-2) / div

    op = operate.astype(jnp.float32)[..., None]
    v = jnp.concatenate([e_c * op, e_c * (1.0 - op)], -1)

    gi = _matmul_bias(v.reshape(BS * L, H), p['gru_Wih'].T, p['gru_bih'], 512)
    gi = gi.reshape(BS, L, 3 * H)

    gi_t = jnp.transpose(gi, (1, 0, 2))
    eq_t = jnp.transpose(e_q, (1, 0, 2))
    ec_t = jnp.transpose(e_c, (1, 0, 2))

    w1 = p['predW1']
    prob = _gru_predict(
        gi_t, eq_t, ec_t,
        p['gru_Whh'].T, p['gru_bhh'],
        w1[:H], w1[H:H + D], w1[H + D:], p['predb1'],
        p['predW2'].reshape(1, D), p['predb2'],
    )
    return prob


# speedup vs baseline: 1.9741x; 1.5015x over previous
import functools

import jax
import jax.numpy as jnp
from jax import lax
from jax.experimental import pallas as pl
from jax.experimental.pallas import tpu as pltpu
from jax.experimental.pallas import tpu_sc as plsc

QN, CN, BERT, D, BS, L, KC, NL = 20000, 2000, 768, 128, 64, 200, 4, 2
H = 2 * D


def _mm_kernel(x_ref, w_ref, b_ref, o_ref):
    o_ref[...] = (
        jnp.dot(x_ref[...], w_ref[...], preferred_element_type=jnp.float32)
        + b_ref[...]
    )


def _matmul_bias(x, w, b, bm):
    m, k = x.shape
    n = w.shape[1]
    assert m % bm == 0, (m, bm)
    return pl.pallas_call(
        _mm_kernel,
        grid=(m // bm,),
        in_specs=[
            pl.BlockSpec((bm, k), lambda i: (i, 0)),
            pl.BlockSpec((k, n), lambda i: (0, 0)),
            pl.BlockSpec((1, n), lambda i: (0, 0)),
        ],
        out_specs=pl.BlockSpec((bm, n), lambda i: (i, 0)),
        out_shape=jax.ShapeDtypeStruct((m, n), jnp.float32),
    )(x, w, b.reshape(1, n))


def _gru_pred_kernel(gi_ref, eq_ref, ec_ref, whh_ref, bhh_ref,
                     w1h_ref, w1q_ref, w1c_ref, b1_ref, w2_ref, b2_ref,
                     o_ref, h_ref):
    t = pl.program_id(0)

    @pl.when(t == 0)
    def _():
        h_ref[...] = jnp.zeros_like(h_ref)

    h = h_ref[...]
    eq = eq_ref[0]
    ec = ec_ref[0]

    px = jnp.tanh(
        jnp.dot(h, w1h_ref[...], preferred_element_type=jnp.float32)
        + jnp.dot(eq, w1q_ref[...], preferred_element_type=jnp.float32)
        + jnp.dot(ec, w1c_ref[...], preferred_element_type=jnp.float32)
        + b1_ref[...]
    )
    logit = jnp.sum(px * w2_ref[...], axis=-1) + b2_ref[0, 0]
    o_ref[0, 0] = jax.nn.sigmoid(logit)

    gi = gi_ref[0]
    gh = jnp.dot(h, whh_ref[...], preferred_element_type=jnp.float32) + bhh_ref[...]
    r = jax.nn.sigmoid(gi[:, :H] + gh[:, :H])
    z = jax.nn.sigmoid(gi[:, H:2 * H] + gh[:, H:2 * H])
    nn = jnp.tanh(gi[:, 2 * H:] + r * gh[:, 2 * H:])
    h_ref[...] = (1.0 - z) * nn + z * h


def _gru_predict(gi, eq, ec, whh_t, bhh, w1h, w1q, w1c, b1, w2row, b2):
    out = pl.pallas_call(
        _gru_pred_kernel,
        grid=(L,),
        in_specs=[
            pl.BlockSpec((1, BS, 3 * H), lambda t: (t, 0, 0)),
            pl.BlockSpec((1, BS, D), lambda t: (t, 0, 0)),
            pl.BlockSpec((1, BS, D), lambda t: (t, 0, 0)),
            pl.BlockSpec((H, 3 * H), lambda t: (0, 0)),
            pl.BlockSpec((1, 3 * H), lambda t: (0, 0)),
            pl.BlockSpec((H, D), lambda t: (0, 0)),
            pl.BlockSpec((D, D), lambda t: (0, 0)),
            pl.BlockSpec((D, D), lambda t: (0, 0)),
            pl.BlockSpec((1, D), lambda t: (0, 0)),
            pl.BlockSpec((1, D), lambda t: (0, 0)),
            pl.BlockSpec((1, 1), lambda t: (0, 0)),
        ],
        out_specs=pl.BlockSpec((1, 1, BS), lambda t: (t, 0, 0)),
        out_shape=jax.ShapeDtypeStruct((L, 1, BS), jnp.float32),
        scratch_shapes=[pltpu.VMEM((BS, H), jnp.float32)],
    )(gi, eq, ec, whh_t, bhh.reshape(1, -1), w1h, w1q, w1c,
      b1.reshape(1, -1), w2row, b2.reshape(1, 1))
    return out.reshape(L, BS).T


def _proj_kernel(x_ref, wm_ref, ws_ref, om_ref, os_ref):
    x = x_ref[...]
    om_ref[...] = jnp.dot(x, wm_ref[...], preferred_element_type=jnp.float32)
    os_ref[...] = jnp.dot(x, ws_ref[...], preferred_element_type=jnp.float32)


def _project(x, wm, ws, bm):
    m, k = x.shape
    nm, ns = wm.shape[1], ws.shape[1]
    assert m % bm == 0
    return pl.pallas_call(
        _proj_kernel,
        grid=(m // bm,),
        in_specs=[
            pl.BlockSpec((bm, k), lambda i: (i, 0)),
            pl.BlockSpec((k, nm), lambda i: (0, 0)),
            pl.BlockSpec((k, ns), lambda i: (0, 0)),
        ],
        out_specs=[
            pl.BlockSpec((bm, nm), lambda i: (i, 0)),
            pl.BlockSpec((bm, ns), lambda i: (i, 0)),
        ],
        out_shape=[
            jax.ShapeDtypeStruct((m, nm), jnp.float32),
            jax.ShapeDtypeStruct((m, ns), jnp.float32),
        ],
    )(x, wm, ws)


_WIDTH = 128


def _gat_agg_sc(xs, ex_edge, src, dst, n_dst, split):
    e_tot = src.shape[0]
    total_groups = e_tot // 16
    if split:
        half = n_dst // 2
        nw = 16
    else:
        half = n_dst
        nw = 32
    out_rows = half
    gpt = -(-total_groups // nw)
    win = gpt * 16
    win += (-win) % 128
    rpt = -(-out_rows // 16)
    rpt += (-rpt) % 8
    mesh = plsc.VectorSubcoreMesh(core_axis_name="c", subcore_axis_name="s")

    @functools.partial(
        pl.kernel, mesh=mesh,
        out_type=jax.ShapeDtypeStruct((2, out_rows, _WIDTH), jnp.float32),
        scratch_types=[
            pltpu.VMEM((win,), jnp.int32),
            pltpu.VMEM((win,), jnp.int32),
            pltpu.VMEM((win,), jnp.float32),
            pltpu.VMEM((16,), jnp.int32),
            pltpu.VMEM((16,), jnp.int32),
            pltpu.VMEM((16, D), jnp.float32),
            pltpu.VMEM((16, _WIDTH), jnp.float32),
            pltpu.VMEM_SHARED((out_rows, _WIDTH), jnp.float32),
            pltpu.SemaphoreType.DMA,
        ],
    )
    def k(xs_hbm, ex_hbm, src_hbm, dst_hbm, zeros_hbm, out_hbm,
          srcw_v, dstw_v, exw_v, sidx_v, didx_v, rows_v,
          outr_v, shared, sem):
        c = lax.axis_index("c")
        s = lax.axis_index("s")
        if split:
            wid = s
            base = c * half
        else:
            wid = s * 2 + c
            base = 0
        start = wid * (gpt * 16)
        start_c = pl.multiple_of(jnp.minimum(start, e_tot - win), 16)
        loff = start - start_c
        pltpu.sync_copy(src_hbm.at[pl.ds(start_c, win)], srcw_v)
        pltpu.sync_copy(dst_hbm.at[pl.ds(start_c, win)], dstw_v)
        pltpu.sync_copy(ex_hbm.at[pl.ds(start_c, win)], exw_v)
        rstart = pl.multiple_of(jnp.minimum(s * rpt, out_rows - rpt), 8)
        pltpu.sync_copy(zeros_hbm.at[pl.ds(rstart, rpt)],
                        shared.at[pl.ds(rstart, rpt)])
        plsc.subcore_barrier()
        lanef = lax.convert_element_type(lax.iota(jnp.int32, 16),
                                         jnp.float32)
        lane0 = jnp.maximum(jnp.sign(0.5 - lanef), 0.0)

        def body(g, carry):
            vfac = jnp.where(wid * gpt + g < total_groups, 1.0, 0.0)
            goff = pl.multiple_of(
                jnp.minimum(loff + g * 16, win - 16), 16)
            src16 = srcw_v[pl.ds(goff, 16)]
            dst16 = dstw_v[pl.ds(goff, 16)]
            exv = exw_v[pl.ds(goff, 16)]
            d0 = dst16 - base
            if split:
                df = lax.convert_element_type(d0, jnp.float32)
                t = df * (df - float(half - 1))
                mf = jnp.maximum(jnp.sign(0.5 - t), 0.0) * vfac
            else:
                mf = jnp.zeros((16,), jnp.float32) + vfac
            exv = exv * mf
            dloc = d0 * lax.convert_element_type(mf, jnp.int32)
            sidx_v[...] = src16
            didx_v[...] = dloc
            pltpu.async_copy(xs_hbm.at[sidx_v], rows_v, sem).wait()
            for j in range(16):
                a = exv[j]
                for kk in range(8):
                    outr_v[j, pl.ds(kk * 16, 16)] = (
                        rows_v[j, pl.ds(kk * 16, 16)] * a)
            pltpu.sync_copy(outr_v, shared.at[didx_v], add=True)
            return carry

        lax.fori_loop(0, gpt, body, 0)
        plsc.subcore_barrier()
        pltpu.sync_copy(shared.at[pl.ds(rstart, rpt)],
                        out_hbm.at[c, pl.ds(rstart, rpt)])

    zeros = jnp.zeros((out_rows, _WIDTH), jnp.float32)
    out = k(xs, ex_edge, src, dst, zeros)
    if split:
        out = out.reshape(n_dst, _WIDTH)
    else:
        out = out[0] + out[1]
    return out


def _gnn(p, g):
    qx, cx = p['q_emb'], p['c_emb']
    eps = 1e-16
    for i in range(NL):
        wm_q = jnp.concatenate([p['qc%d_Wsrc' % i], p['linWq%d' % i]], 1)
        ws_q = jnp.stack([
            p['qc%d_Wsrc' % i] @ p['qc%d_asrc' % i],
            p['cq%d_Wdst' % i] @ p['cq%d_adst' % i],
        ], 1)
        ws_q = jnp.pad(ws_q, ((0, 0), (0, 6)))
        wm_c = jnp.concatenate(
            [p['cc%d_Wsrc' % i], p['cq%d_Wsrc' % i], p['linWc%d' % i]], 1)
        ws_c = jnp.stack([
            p['cc%d_Wsrc' % i] @ p['cc%d_asrc' % i],
            p['cc%d_Wdst' % i] @ p['cc%d_adst' % i],
            p['qc%d_Wdst' % i] @ p['qc%d_adst' % i],
            p['cq%d_Wsrc' % i] @ p['cq%d_asrc' % i],
        ], 1)
        ws_c = jnp.pad(ws_c, ((0, 0), (0, 4)))

        pq_m, pq_s = _project(qx, wm_q, ws_q, 400)
        pc_m, pc_s = _project(cx, wm_c, ws_c, 400)
        qc_xs, lin_q = pq_m[:, :D], pq_m[:, D:]
        es_qc, ed_cq = pq_s[:, 0], pq_s[:, 1]
        cc_xs, cq_xs, lin_c = pc_m[:, :D], pc_m[:, D:2 * D], pc_m[:, 2 * D:]
        es_cc, ed_cc = pc_s[:, 0], pc_s[:, 1]
        ed_qc, es_cq = pc_s[:, 2], pc_s[:, 3]

        ex_cc = jnp.exp(jax.nn.leaky_relu(
            es_cc[g['cc_src']] + ed_cc[g['cc_dst']], 0.2))
        ex_qc = jnp.exp(jax.nn.leaky_relu(
            es_qc[g['qc_src']] + ed_qc[g['qc_dst']], 0.2))
        ex_cq = jnp.exp(jax.nn.leaky_relu(
            es_cq[g['cq_src']] + ed_cq[g['cq_dst']], 0.2))
        n_cc = _gat_agg_sc(cc_xs, ex_cc,
                           g['cc_src'], g['cc_dst'], CN, split=False)
        n_qc = _gat_agg_sc(qc_xs, ex_qc,
                           g['qc_src'], g['qc_dst'], CN, split=False)
        n_cq = _gat_agg_sc(cq_xs, ex_cq,
                           g['cq_src'], g['cq_dst'], QN, split=True)
        s_cc = jax.ops.segment_sum(ex_cc, g['cc_dst'], num_segments=CN)
        s_qc = jax.ops.segment_sum(ex_qc, g['qc_dst'], num_segments=CN)
        s_cq = jax.ops.segment_sum(ex_cq, g['cq_dst'], num_segments=QN)

        cx = jax.nn.relu(n_cc / (s_cc[:, None] + eps) + p['cc%d_b' % i]
                         + n_qc / (s_qc[:, None] + eps) + p['qc%d_b' % i]
                         + lin_c + p['linbc%d' % i])
        qx = jax.nn.relu(n_cq / (s_cq[:, None] + eps) + p['cq%d_b' % i]
                         + lin_q + p['linbq%d' % i])
    return qx, cx


def kernel(q_seq, l_seq, concepts, operate, btype, text, text1, text2, params, graph):
    p = params
    qx, cx = _gnn(p, graph)

    q_tab = jnp.concatenate([jnp.zeros((1, D), qx.dtype), qx], 0)
    c_tab = jnp.concatenate([jnp.zeros((1, D), cx.dtype), cx], 0)
    e_q = q_tab[q_seq]
    e_c_m = c_tab[concepts]
    filt = (concepts != 0).astype(jnp.float32)
    fs = filt.sum(-1)
    div = jnp.where(fs == 0, 1.0, fs)[..., None]
    e_c = e_c_m.sum(-2) / div

    op = operate.astype(jnp.float32)[..., None]
    v = jnp.concatenate([e_c * op, e_c * (1.0 - op)], -1)

    gi = _matmul_bias(v.reshape(BS * L, H), p['gru_Wih'].T, p['gru_bih'], 512)
    gi = gi.reshape(BS, L, 3 * H)

    gi_t = jnp.transpose(gi, (1, 0, 2))
    eq_t = jnp.transpose(e_q, (1, 0, 2))
    ec_t = jnp.transpose(e_c, (1, 0, 2))

    w1 = p['predW1']
    prob = _gru_predict(
        gi_t, eq_t, ec_t,
        p['gru_Whh'].T, p['gru_bhh'],
        w1[:H], w1[H:H + D], w1[H + D:], p['predb1'],
        p['predW2'].reshape(1, D), p['predb2'],
    )
    return prob
